# baseline (speedup 1.0000x reference)
import functools

import jax
import jax.numpy as jnp
from jax import lax
from jax.experimental import pallas as pl
from jax.experimental.pallas import tpu as pltpu
from jax.experimental.pallas import tpu_sc as plsc

_N_FIELDS = 26
_EMBED_DIM = 64
_BATCH = 4096
_TOTAL = _BATCH * _N_FIELDS
_NC, _NS, _LANES = 2, 16, 16
_NW = _NC * _NS
_SAMP_W = _BATCH // _NW
_SPB = 4
_ROW = _SPB * _N_FIELDS
_G = _SAMP_W // _SPB
_NB = 4

_mesh = plsc.VectorSubcoreMesh(core_axis_name="c", subcore_axis_name="s")


@functools.partial(
    pl.kernel,
    mesh=_mesh,
    out_type=jax.ShapeDtypeStruct((_BATCH, _N_FIELDS, _EMBED_DIM),
                                  jnp.float32),
    scratch_types=[
        pltpu.VMEM((_NW * _ROW,), jnp.int32),
        pltpu.VMEM((_NB, _ROW, _EMBED_DIM), jnp.float32),
        [pltpu.SemaphoreType.DMA] * _NB,
        [pltpu.SemaphoreType.DMA] * _NB,
    ],
    compiler_params=pltpu.CompilerParams(use_tc_tiling_on_sc=False),
)
def _emb_lookup(x_hbm, table_hbm, out_hbm, idx_v, rows_v, gsems, wsems):
    wid = lax.axis_index("s") * _NC + lax.axis_index("c")
    base = wid * _NW * _ROW
    base_samp = wid * _SAMP_W

    lane = lax.iota(jnp.int32, _LANES)
    pats = {
        s: ((s + lane) % _N_FIELDS) * 1000 for s in range(0, _N_FIELDS, 2)
    }

    pltpu.sync_copy(x_hbm.at[pl.ds(base, _NW * _ROW)], idx_v)
    for j in range(_NW * _ROW // _LANES):
        s = pl.ds(j * _LANES, _LANES)
        idx_v[s] = idx_v[s] + pats[(j * _LANES) % _N_FIELDS]

    def gather(g, b):
        return pltpu.async_copy(
            table_hbm.at[idx_v.at[pl.ds(g * _ROW, _ROW)]], rows_v.at[b],
            gsems[b])

    def write(g, b):
        return [
            pltpu.async_copy(
                rows_v.at[b].at[pl.ds(i * _N_FIELDS, _N_FIELDS)],
                out_hbm.at[base_samp + g * _SPB + i],
                wsems[b])
            for i in range(_SPB)
        ]

    hg = [None] * _NB
    hw = [[] for _ in range(_NB)]
    for b in range(_NB):
        hg[b] = gather(b, b)
    for g in range(_G):
        b = g % _NB
        hg[b].wait()
        hw[b] = write(g, b)
        nxt = g + _NB
        if nxt < _G:
            for h in hw[b]:
                h.wait()
            hg[b] = gather(nxt, b)
    for g in range(max(0, _G - _NB), _G):
        for h in hw[g % _NB]:
            h.wait()


def kernel(x, table):
    return _emb_lookup(x.reshape(_TOTAL), table)


# speedup vs baseline: 1.2353x; 1.2353x over previous
import functools

import jax
import jax.experimental.layout
import jax.numpy as jnp
from jax import lax
from jax.experimental import pallas as pl
from jax.experimental.pallas import tpu as pltpu
from jax.experimental.pallas import tpu_sc as plsc

_N_FIELDS = 26
_EMBED_DIM = 64
_BATCH = 4096
_TOTAL = _BATCH * _N_FIELDS
_NC, _NS, _LANES = 2, 16, 16
_NW = _NC * _NS
_SAMP_W = _BATCH // _NW
_SPB = 4
_ROW = _SPB * _N_FIELDS
_G = _SAMP_W // _SPB
_NB = 4

_mesh = plsc.VectorSubcoreMesh(core_axis_name="c", subcore_axis_name="s")


@functools.partial(
    pl.kernel,
    mesh=_mesh,
    out_type=jax.ShapeDtypeStruct((_BATCH, _N_FIELDS, _EMBED_DIM),
                                  jnp.float32),
    scratch_types=[
        pltpu.VMEM((_NW * _ROW,), jnp.int32),
        pltpu.VMEM((_NB, _ROW, _EMBED_DIM), jnp.float32),
        [pltpu.SemaphoreType.DMA] * _NB,
        [pltpu.SemaphoreType.DMA] * _NB,
    ],
    compiler_params=pltpu.CompilerParams(use_tc_tiling_on_sc=False),
)
def _emb_lookup(x_hbm, table_hbm, out_hbm, idx_v, rows_v, gsems, wsems):
    wid = lax.axis_index("s") * _NC + lax.axis_index("c")
    base = wid * _NW * _ROW
    base_samp = wid * _SAMP_W

    lane = lax.iota(jnp.int32, _LANES)
    pats = {
        s: ((s + lane) % _N_FIELDS) * 1000 for s in range(0, _N_FIELDS, 2)
    }

    pltpu.sync_copy(x_hbm.at[pl.ds(base, _NW * _ROW)], idx_v)
    for j in range(_NW * _ROW // _LANES):
        s = pl.ds(j * _LANES, _LANES)
        idx_v[s] = idx_v[s] + pats[(j * _LANES) % _N_FIELDS]

    def gather(g, b):
        return pltpu.async_copy(
            table_hbm.at[idx_v.at[pl.ds(g * _ROW, _ROW)]], rows_v.at[b],
            gsems[b])

    def write(g, b):
        return [
            pltpu.async_copy(
                rows_v.at[b].at[pl.ds(i * _N_FIELDS, _N_FIELDS)],
                out_hbm.at[base_samp + g * _SPB + i],
                wsems[b])
            for i in range(_SPB)
        ]

    hg = [None] * _NB
    hw = [[] for _ in range(_NB)]
    for b in range(_NB):
        hg[b] = gather(b, b)
    for g in range(_G):
        b = g % _NB
        hg[b].wait()
        hw[b] = write(g, b)
        nxt = g + _NB
        if nxt < _G:
            for h in hw[b]:
                h.wait()
            hg[b] = gather(nxt, b)
    for g in range(max(0, _G - _NB), _G):
        for h in hw[g % _NB]:
            h.wait()


def kernel(x, table):
    out = _emb_lookup(x.reshape(_TOTAL), table)
    return jax.experimental.layout.with_layout_constraint(
        out, jax.experimental.layout.Layout(major_to_minor=(0, 1, 2)))
